# baseline (device time: 7901 ns/iter reference)
import jax
import jax.numpy as jnp
from jax import lax
from jax.experimental import pallas as pl
from jax.experimental.pallas import tpu as pltpu

_CDTYPE = jnp.bfloat16


def kernel(u):
    nx, ny, nz = u.shape

    def body(u_ref, out_ref, send_x, send_y, send_z,
             recv_x, recv_y, recv_z, send_sems, recv_sems, ready_sems):
        mx = lax.axis_index("x")
        my = lax.axis_index("y")
        mz = lax.axis_index("z")

        nbr_x = (1 - mx, my, mz)
        nbr_y = (mx, 1 - my, mz)
        nbr_z = (mx, my, 1 - mz)
        neighbors = (nbr_x, nbr_y, nbr_z)

        barrier = pltpu.get_barrier_semaphore()
        pl.semaphore_signal(barrier, inc=1, device_id=nbr_x,
                            device_id_type=pl.DeviceIdType.MESH)
        for axis, nbr in ((1, nbr_y), (2, nbr_z)):
            pl.semaphore_signal(ready_sems.at[axis], inc=1, device_id=nbr,
                                device_id_type=pl.DeviceIdType.MESH)

        send_x[...] = jnp.where(mx == 0, u_ref[nx - 1, :, :],
                                u_ref[0, :, :]).astype(_CDTYPE)
        send_y[...] = jnp.where(my == 0, u_ref[:, ny - 1, :],
                                u_ref[:, 0, :]).astype(_CDTYPE)
        send_z[...] = jnp.where(mz == 0, u_ref[:, :, nz - 1],
                                u_ref[:, :, 0]).astype(_CDTYPE)

        rdmas = []
        for axis, (sbuf, rbuf, nbr) in enumerate(
            ((send_x, recv_x, nbr_x),
             (send_y, recv_y, nbr_y),
             (send_z, recv_z, nbr_z))
        ):
            if axis == 0:
                pl.semaphore_wait(barrier, 1)
            else:
                pl.semaphore_wait(ready_sems.at[axis], 1)
            rdma = pltpu.make_async_remote_copy(
                src_ref=sbuf,
                dst_ref=rbuf,
                send_sem=send_sems.at[axis],
                recv_sem=recv_sems.at[axis],
                device_id=nbr,
                device_id_type=pl.DeviceIdType.MESH,
            )
            rdma.start()
            rdmas.append(rdma)

        u_val = u_ref[...].astype(_CDTYPE)
        zx = jnp.zeros((1, ny, nz), _CDTYPE)
        zy = jnp.zeros((nx, 1, nz), _CDTYPE)
        zz = jnp.zeros((nx, ny, 1), _CDTYPE)
        v = (
            jnp.concatenate([u_val[1:, :, :], zx], axis=0)
            + jnp.concatenate([zx, u_val[:-1, :, :]], axis=0)
            + jnp.concatenate([u_val[:, 1:, :], zy], axis=1)
            + jnp.concatenate([zy, u_val[:, :-1, :]], axis=1)
            + jnp.concatenate([u_val[:, :, 1:], zz], axis=2)
            + jnp.concatenate([zz, u_val[:, :, :-1]], axis=2)
            - 6.0 * u_val
        )

        iy = jnp.where(my == 0, ny - 1, 0)
        iz = jnp.where(mz == 0, nz - 1, 0)
        ix = jnp.where(mx == 0, nx - 1, 0)
        i0 = lax.broadcasted_iota(jnp.int32, (nx, ny, nz), 0)
        i1 = lax.broadcasted_iota(jnp.int32, (nx, ny, nz), 1)
        i2 = lax.broadcasted_iota(jnp.int32, (nx, ny, nz), 2)
        sel_y = (i1 == iy).astype(_CDTYPE)
        sel_z = (i2 == iz).astype(_CDTYPE)
        bad = (
            ((mx == 0) & (i0 == 0)) | ((mx == 1) & (i0 == nx - 1))
            | ((my == 0) & (i1 == 0)) | ((my == 1) & (i1 == ny - 1))
            | ((mz == 0) & (i2 == 0)) | ((mz == 1) & (i2 == nz - 1))
        )
        jx = lax.broadcasted_iota(jnp.int32, (ny, nz), 0)
        kx = lax.broadcasted_iota(jnp.int32, (ny, nz), 1)
        edge_x = ~(
            ((my == 0) & (jx == 0)) | ((my == 1) & (jx == ny - 1))
            | ((mz == 0) & (kx == 0)) | ((mz == 1) & (kx == nz - 1))
        )

        rdmas[1].wait()
        v = v + sel_y * recv_y[...][:, None, :]
        rdmas[2].wait()
        v = v + sel_z * recv_z[...][:, :, None]

        out_ref[...] = jnp.where(bad, jnp.zeros_like(v), v)

        rdmas[0].wait()
        px = jnp.where(edge_x, recv_x[...], jnp.zeros_like(recv_x[...]))
        out_ref[pl.ds(ix, 1), :, :] = out_ref[pl.ds(ix, 1), :, :] + px[None]

    return pl.pallas_call(
        body,
        out_shape=jax.ShapeDtypeStruct((nx, ny, nz), _CDTYPE),
        in_specs=[pl.BlockSpec(memory_space=pltpu.VMEM)],
        out_specs=pl.BlockSpec(memory_space=pltpu.VMEM),
        scratch_shapes=[
            pltpu.VMEM((ny, nz), _CDTYPE),
            pltpu.VMEM((nx, nz), _CDTYPE),
            pltpu.VMEM((nx, ny), _CDTYPE),
            pltpu.VMEM((ny, nz), _CDTYPE),
            pltpu.VMEM((nx, nz), _CDTYPE),
            pltpu.VMEM((nx, ny), _CDTYPE),
            pltpu.SemaphoreType.DMA((3,)),
            pltpu.SemaphoreType.DMA((3,)),
            pltpu.SemaphoreType.REGULAR((3,)),
        ],
        compiler_params=pltpu.CompilerParams(collective_id=0),
    )(u)


# device time: 7828 ns/iter; 1.0093x vs baseline; 1.0093x over previous
import jax
import jax.numpy as jnp
from jax import lax
from jax.experimental import pallas as pl
from jax.experimental.pallas import tpu as pltpu

_CDTYPE = jnp.bfloat16


def kernel(u):
    nx, ny, nz = u.shape

    def body(u_ref, out_ref, send_x, send_y, send_z,
             recv_x, recv_y, recv_z, send_sems, recv_sems):
        mx = lax.axis_index("x")
        my = lax.axis_index("y")
        mz = lax.axis_index("z")

        nbr_x = (1 - mx, my, mz)
        nbr_y = (mx, 1 - my, mz)
        nbr_z = (mx, my, 1 - mz)
        neighbors = (nbr_x, nbr_y, nbr_z)

        barrier = pltpu.get_barrier_semaphore()
        for nbr in neighbors:
            pl.semaphore_signal(barrier, inc=1, device_id=nbr,
                                device_id_type=pl.DeviceIdType.MESH)

        send_x[...] = jnp.where(mx == 0, u_ref[nx - 1, :, :],
                                u_ref[0, :, :]).astype(_CDTYPE)
        send_y[...] = jnp.where(my == 0, u_ref[:, ny - 1, :],
                                u_ref[:, 0, :]).astype(_CDTYPE)
        send_z[...] = jnp.where(mz == 0, u_ref[:, :, nz - 1],
                                u_ref[:, :, 0]).astype(_CDTYPE)

        pl.semaphore_wait(barrier, 3)

        rdmas = []
        for axis, (sbuf, rbuf, nbr) in enumerate(
            ((send_x, recv_x, nbr_x),
             (send_y, recv_y, nbr_y),
             (send_z, recv_z, nbr_z))
        ):
            rdma = pltpu.make_async_remote_copy(
                src_ref=sbuf,
                dst_ref=rbuf,
                send_sem=send_sems.at[axis],
                recv_sem=recv_sems.at[axis],
                device_id=nbr,
                device_id_type=pl.DeviceIdType.MESH,
            )
            rdma.start()
            rdmas.append(rdma)

        u_val = u_ref[...].astype(_CDTYPE)
        zx = jnp.zeros((1, ny, nz), _CDTYPE)
        zy = jnp.zeros((nx, 1, nz), _CDTYPE)
        zz = jnp.zeros((nx, ny, 1), _CDTYPE)
        v = (
            jnp.concatenate([u_val[1:, :, :], zx], axis=0)
            + jnp.concatenate([zx, u_val[:-1, :, :]], axis=0)
            + jnp.concatenate([u_val[:, 1:, :], zy], axis=1)
            + jnp.concatenate([zy, u_val[:, :-1, :]], axis=1)
            + jnp.concatenate([u_val[:, :, 1:], zz], axis=2)
            + jnp.concatenate([zz, u_val[:, :, :-1]], axis=2)
            - 6.0 * u_val
        )

        iy = jnp.where(my == 0, ny - 1, 0)
        iz = jnp.where(mz == 0, nz - 1, 0)
        ix = jnp.where(mx == 0, nx - 1, 0)
        i0 = lax.broadcasted_iota(jnp.int32, (nx, ny, nz), 0)
        i1 = lax.broadcasted_iota(jnp.int32, (nx, ny, nz), 1)
        i2 = lax.broadcasted_iota(jnp.int32, (nx, ny, nz), 2)
        sel_y = (i1 == iy).astype(_CDTYPE)
        sel_z = (i2 == iz).astype(_CDTYPE)
        bad = (
            ((mx == 0) & (i0 == 0)) | ((mx == 1) & (i0 == nx - 1))
            | ((my == 0) & (i1 == 0)) | ((my == 1) & (i1 == ny - 1))
            | ((mz == 0) & (i2 == 0)) | ((mz == 1) & (i2 == nz - 1))
        )
        jx = lax.broadcasted_iota(jnp.int32, (ny, nz), 0)
        kx = lax.broadcasted_iota(jnp.int32, (ny, nz), 1)
        edge_x = ~(
            ((my == 0) & (jx == 0)) | ((my == 1) & (jx == ny - 1))
            | ((mz == 0) & (kx == 0)) | ((mz == 1) & (kx == nz - 1))
        )

        rdmas[1].wait()
        v = v + sel_y * recv_y[...][:, None, :]
        rdmas[2].wait()
        v = v + sel_z * recv_z[...][:, :, None]

        out_ref[...] = jnp.where(bad, jnp.zeros_like(v), v)

        rdmas[0].wait()
        px = jnp.where(edge_x, recv_x[...], jnp.zeros_like(recv_x[...]))
        out_ref[pl.ds(ix, 1), :, :] = out_ref[pl.ds(ix, 1), :, :] + px[None]

    return pl.pallas_call(
        body,
        out_shape=jax.ShapeDtypeStruct((nx, ny, nz), _CDTYPE),
        in_specs=[pl.BlockSpec(memory_space=pltpu.VMEM)],
        out_specs=pl.BlockSpec(memory_space=pltpu.VMEM),
        scratch_shapes=[
            pltpu.VMEM((ny, nz), _CDTYPE),
            pltpu.VMEM((nx, nz), _CDTYPE),
            pltpu.VMEM((nx, ny), _CDTYPE),
            pltpu.VMEM((ny, nz), _CDTYPE),
            pltpu.VMEM((nx, nz), _CDTYPE),
            pltpu.VMEM((nx, ny), _CDTYPE),
            pltpu.SemaphoreType.DMA((3,)),
            pltpu.SemaphoreType.DMA((3,)),
        ],
        compiler_params=pltpu.CompilerParams(collective_id=0),
    )(u)
